# baseline (speedup 1.0000x reference)
import functools

import jax
import jax.numpy as jnp
from jax import lax
from jax.experimental import pallas as pl
from jax.experimental.pallas import tpu as pltpu
from jax.experimental.pallas import tpu_sc as plsc

_T = 4096
_D = 1024
_E = 8
_K = 2

_BT = 128
_S = _K * _T + _E * _BT
_NB = _S // _BT

_NW = 32
_TPW = _T // _NW

_DCH = 32
_DNC = _TPW // _DCH
_CCH = 16
_CNC = _TPW // _CCH


def _gate_kernel(x_ref, gW1_ref, gb1_ref, gW2_ref, gb2_ref,
                 g0_ref, g1_ref, p0_ref, p1_ref, eid_ref, aux_ref, xp_ref):
    x = x_ref[...]
    z = jnp.tanh(
        lax.dot_general(x, gW1_ref[...], (((1,), (0,)), ((), ())))
        + gb1_ref[...])
    logits = (lax.dot_general(z, gW2_ref[...], (((1,), (0,)), ((), ())))
              + gb2_ref[...])
    m = jnp.max(logits, axis=-1, keepdims=True)
    ex = jnp.exp(logits - m)
    scores = ex / jnp.sum(ex, axis=-1, keepdims=True)

    eidx = lax.broadcasted_iota(jnp.int32, (_T, _E), 1)
    m0 = jnp.max(scores, axis=-1, keepdims=True)
    i0 = jnp.min(jnp.where(scores == m0, eidx, _E), axis=-1, keepdims=True)
    masked = jnp.where(eidx == i0, -jnp.inf, scores)
    m1 = jnp.max(masked, axis=-1, keepdims=True)
    i1 = jnp.min(jnp.where(masked == m1, eidx, _E), axis=-1, keepdims=True)

    denom = m0 + m1 + 1e-6
    g0_ref[...] = jnp.broadcast_to(m0 / denom, (_T, 16))
    g1_ref[...] = jnp.broadcast_to(m1 / denom, (_T, 16))

    oh0 = (eidx == i0).astype(jnp.float32)
    oh1 = (eidx == i1).astype(jnp.float32)
    ind = oh0 + oh1

    importance = jnp.sum(scores, axis=0, keepdims=True)
    load = jnp.sum(ind, axis=0, keepdims=True)
    aux_ref[...] = (_E / (_T * _T)) * jnp.sum(importance * load, axis=1,
                                              keepdims=True)

    c = ind
    k = 1
    while k < _T:
        c = c + jnp.concatenate([jnp.zeros((k, _E), jnp.float32), c[:-k, :]],
                                axis=0)
        k *= 2
    exc = c - ind
    cnt = c[_T - 1:_T, :]
    pad_cnt = jnp.ceil(cnt * (1.0 / _BT)) * _BT
    tri = (lax.broadcasted_iota(jnp.int32, (_E, _E), 0)
           < lax.broadcasted_iota(jnp.int32, (_E, _E), 1)).astype(jnp.float32)
    off = lax.dot_general(pad_cnt, tri, (((1,), (0,)), ((), ())))

    pos = off + exc
    p0_ref[...] = jnp.sum(oh0 * pos, axis=1, keepdims=True).astype(jnp.int32)
    p1_ref[...] = jnp.sum(oh1 * pos, axis=1, keepdims=True).astype(jnp.int32)

    bstart = (lax.broadcasted_iota(jnp.int32, (_NB, 1), 0) * _BT
              ).astype(jnp.float32)
    ge = (bstart >= off).astype(jnp.int32)
    eid_ref[...] = jnp.sum(ge, axis=1, keepdims=True) - 1

    xb16 = x.astype(jnp.bfloat16)
    lo = lax.bitcast_convert_type(xb16[:, :_D // 2], jnp.uint16)
    hi = lax.bitcast_convert_type(xb16[:, _D // 2:], jnp.uint16)
    xp_ref[...] = ((hi.astype(jnp.int32) << 16) | lo.astype(jnp.int32))


def _gate(x, gW1, gb1, gW2, gb2):
    out_shape = (
        jax.ShapeDtypeStruct((_T, 16), jnp.float32),
        jax.ShapeDtypeStruct((_T, 16), jnp.float32),
        jax.ShapeDtypeStruct((_T, 1), jnp.int32),
        jax.ShapeDtypeStruct((_T, 1), jnp.int32),
        jax.ShapeDtypeStruct((_NB, 1), jnp.int32),
        jax.ShapeDtypeStruct((1, 1), jnp.float32),
        jax.ShapeDtypeStruct((_T, _D // 2), jnp.int32),
    )
    return pl.pallas_call(
        _gate_kernel,
        out_shape=out_shape,
    )(x, gW1, gb1.reshape(1, -1), gW2, gb2.reshape(1, -1))


def _dispatch_body(x_hbm, idx_hbm, xs_hbm,
                   idx_v, buf_a, buf_b, sem_a, sem_b):
    wid = lax.axis_index("s") * 2 + lax.axis_index("c")
    base = wid * _TPW
    pltpu.sync_copy(idx_hbm.at[wid], idx_v)
    bufs = (buf_a, buf_b)
    sems = (sem_a, sem_b)
    pend = [None, None]
    for c in range(_DNC):
        p = c % 2
        if pend[p] is not None:
            pend[p][0].wait()
            pend[p][1].wait()
        pltpu.sync_copy(x_hbm.at[pl.ds(base + c * _DCH, _DCH)], bufs[p])
        cp0 = pltpu.async_copy(bufs[p], xs_hbm.at[idx_v.at[c]], sems[p])
        cp1 = pltpu.async_copy(bufs[p], xs_hbm.at[idx_v.at[_DNC + c]],
                               sems[p])
        pend[p] = (cp0, cp1)
    for p in range(2):
        if pend[p] is not None:
            pend[p][0].wait()
            pend[p][1].wait()


def _dispatch(x, idx4):
    mesh = plsc.VectorSubcoreMesh(core_axis_name="c", subcore_axis_name="s")
    f = pl.kernel(
        _dispatch_body,
        out_type=jax.ShapeDtypeStruct((_S, _D // 2), jnp.int32),
        mesh=mesh,
        scratch_types=[
            pltpu.VMEM((2 * _DNC, _DCH), jnp.int32),
            pltpu.VMEM((_DCH, _D // 2), jnp.int32),
            pltpu.VMEM((_DCH, _D // 2), jnp.int32),
            pltpu.SemaphoreType.DMA,
            pltpu.SemaphoreType.DMA,
        ],
    )
    return f(x, idx4)


def _group_kernel(eid_ref, xs_ref, Wa_ref, ba_ref, lg_ref, lb_ref,
                  Wb_ref, bb_ref, os_ref, wa_s, wb_s, last_s):
    i = pl.program_id(0)

    @pl.when(i == 0)
    def _():
        last_s[0] = -1

    e = eid_ref[i]

    @pl.when(last_s[0] != e)
    def _():
        wa_s[...] = Wa_ref[0].astype(jnp.bfloat16)
        wb_s[...] = Wb_ref[0].astype(jnp.bfloat16)
        last_s[0] = e

    w = xs_ref[...]
    xlo = lax.bitcast_convert_type((w & 0xFFFF).astype(jnp.uint16),
                                   jnp.bfloat16)
    xhi = lax.bitcast_convert_type(
        lax.shift_right_logical(w, 16).astype(jnp.uint16), jnp.bfloat16)
    xb = jnp.concatenate([xlo, xhi], axis=1)
    h = lax.dot_general(xb, wa_s[...], (((1,), (0,)), ((), ())),
                        preferred_element_type=jnp.float32) + ba_ref[0]
    mu = jnp.mean(h, axis=-1, keepdims=True)
    var = jnp.mean((h - mu) ** 2, axis=-1, keepdims=True)
    hn = (h - mu) * lax.rsqrt(var + 1e-5) * lg_ref[0] + lb_ref[0]
    ho = jnp.maximum(hn, 0.0).astype(jnp.bfloat16)
    eo = lax.dot_general(ho, wb_s[...], (((1,), (0,)), ((), ())),
                         preferred_element_type=jnp.float32) + bb_ref[0]
    eb = eo.astype(jnp.bfloat16)
    olo = lax.bitcast_convert_type(eb[:, :_D // 2], jnp.uint16)
    ohi = lax.bitcast_convert_type(eb[:, _D // 2:], jnp.uint16)
    os_ref[...] = (ohi.astype(jnp.int32) << 16) | olo.astype(jnp.int32)


def _grouped(eid, xs, Wa_bf, ba3, lg3, lb3, Wb_bf, bb3):
    grid_spec = pltpu.PrefetchScalarGridSpec(
        num_scalar_prefetch=1,
        grid=(_NB,),
        in_specs=[
            pl.BlockSpec((_BT, _D // 2), lambda i, eid: (i, 0)),
            pl.BlockSpec((1, _D, _D), lambda i, eid: (eid[i], 0, 0)),
            pl.BlockSpec((1, 1, _D), lambda i, eid: (eid[i], 0, 0)),
            pl.BlockSpec((1, 1, _D), lambda i, eid: (eid[i], 0, 0)),
            pl.BlockSpec((1, 1, _D), lambda i, eid: (eid[i], 0, 0)),
            pl.BlockSpec((1, _D, _D), lambda i, eid: (eid[i], 0, 0)),
            pl.BlockSpec((1, 1, _D), lambda i, eid: (eid[i], 0, 0)),
        ],
        out_specs=pl.BlockSpec((_BT, _D // 2), lambda i, eid: (i, 0)),
        scratch_shapes=[
            pltpu.VMEM((_D, _D), jnp.bfloat16),
            pltpu.VMEM((_D, _D), jnp.bfloat16),
            pltpu.SMEM((1,), jnp.int32),
        ],
    )
    return pl.pallas_call(
        _group_kernel,
        grid_spec=grid_spec,
        out_shape=jax.ShapeDtypeStruct((_S, _D // 2), jnp.int32),
    )(eid, xs, Wa_bf, ba3, lg3, lb3, Wb_bf, bb3)


def _combine_body(os_hbm, idx_hbm, g0_hbm, g1_hbm, out_hbm,
                  idx_v, g0_v, g1_v, b0a, b1a, b0b, b1b, ob,
                  s0a, s1a, s0b, s1b):
    wid = lax.axis_index("s") * 2 + lax.axis_index("c")
    base = wid * _TPW
    pltpu.sync_copy(idx_hbm.at[wid], idx_v)
    pltpu.sync_copy(g0_hbm.at[pl.ds(base, _TPW)], g0_v)
    pltpu.sync_copy(g1_hbm.at[pl.ds(base, _TPW)], g1_v)
    b0 = (b0a, b0b)
    b1 = (b1a, b1b)
    s0 = (s0a, s0b)
    s1 = (s1a, s1b)

    def gathers(c, p):
        cp0 = pltpu.async_copy(os_hbm.at[idx_v.at[c]], b0[p], s0[p])
        cp1 = pltpu.async_copy(os_hbm.at[idx_v.at[_CNC + c]], b1[p], s1[p])
        return cp0, cp1

    pend = gathers(0, 0)
    for c in range(_CNC):
        p = c % 2
        nxt = gathers(c + 1, 1 - p) if c + 1 < _CNC else None
        pend[0].wait()
        pend[1].wait()

        bb0 = b0[p]
        bb1 = b1[p]

        tok0 = c * _CCH

        def body(r, _):
            a = g0_v[tok0 + r]
            b = g1_v[tok0 + r]
            for j in range(_D // 32):
                sl = pl.ds(j * 16, 16)
                w0 = bb0[r, sl]
                w1 = bb1[r, sl]
                lo0 = lax.bitcast_convert_type(w0 << 16, jnp.float32)
                hi0 = lax.bitcast_convert_type(w0 & -65536, jnp.float32)
                lo1 = lax.bitcast_convert_type(w1 << 16, jnp.float32)
                hi1 = lax.bitcast_convert_type(w1 & -65536, jnp.float32)
                ob[r, pl.ds(j * 16, 16)] = a * lo0 + b * lo1
                ob[r, pl.ds(_D // 2 + j * 16, 16)] = a * hi0 + b * hi1
            return 0

        lax.fori_loop(0, _CCH, body, 0)
        pltpu.sync_copy(ob, out_hbm.at[pl.ds(base + c * _CCH, _CCH)])
        pend = nxt


def _combine(os, idx4c, g0r, g1r):
    mesh = plsc.VectorSubcoreMesh(core_axis_name="c", subcore_axis_name="s")
    f = pl.kernel(
        _combine_body,
        out_type=jax.ShapeDtypeStruct((_T, _D), jnp.float32),
        mesh=mesh,
        scratch_types=[
            pltpu.VMEM((2 * _CNC, _CCH), jnp.int32),
            pltpu.VMEM((_TPW, 16), jnp.float32),
            pltpu.VMEM((_TPW, 16), jnp.float32),
            pltpu.VMEM((_CCH, _D // 2), jnp.int32),
            pltpu.VMEM((_CCH, _D // 2), jnp.int32),
            pltpu.VMEM((_CCH, _D // 2), jnp.int32),
            pltpu.VMEM((_CCH, _D // 2), jnp.int32),
            pltpu.VMEM((_CCH, _D), jnp.float32),
            pltpu.SemaphoreType.DMA,
            pltpu.SemaphoreType.DMA,
            pltpu.SemaphoreType.DMA,
            pltpu.SemaphoreType.DMA,
        ],
    )
    return f(os, idx4c, g0r, g1r)


def kernel(x, gW1, gb1, gW2, gb2, Wa, ba, lg, lb, Wb, bb):
    g0, g1, pos0, pos1, eid, aux, xp = _gate(x, gW1, gb1, gW2, gb2)

    idx4 = jnp.concatenate(
        [pos0.reshape(_NW, _DNC, _DCH), pos1.reshape(_NW, _DNC, _DCH)],
        axis=1)
    idx4c = jnp.concatenate(
        [pos0.reshape(_NW, _CNC, _CCH), pos1.reshape(_NW, _CNC, _CCH)],
        axis=1)

    xs = _dispatch(xp, idx4)
    os = _grouped(eid.reshape(_NB), xs,
                  Wa, ba.reshape(_E, 1, _D),
                  lg.reshape(_E, 1, _D), lb.reshape(_E, 1, _D),
                  Wb, bb.reshape(_E, 1, _D))
    out = _combine(os, idx4c, g0, g1)
    return out, aux[0, 0]


# speedup vs baseline: 1.0670x; 1.0670x over previous
import functools

import jax
import jax.numpy as jnp
from jax import lax
from jax.experimental import pallas as pl
from jax.experimental.pallas import tpu as pltpu
from jax.experimental.pallas import tpu_sc as plsc

_T = 4096
_D = 1024
_E = 8
_K = 2

_BT = 128
_S = _K * _T + _E * _BT
_NB = _S // _BT

_NW = 32
_TPW = _T // _NW

_DCH = 32
_DNC = _TPW // _DCH
_CCH = 16
_CNC = _TPW // _CCH


def _gate_kernel(x_ref, gW1_ref, gb1_ref, gW2_ref, gb2_ref,
                 g0_ref, g1_ref, p0_ref, p1_ref, eid_ref, aux_ref, xp_ref):
    x = x_ref[...]
    z = jnp.tanh(
        lax.dot_general(x, gW1_ref[...], (((1,), (0,)), ((), ())))
        + gb1_ref[...])
    logits = (lax.dot_general(z, gW2_ref[...], (((1,), (0,)), ((), ())))
              + gb2_ref[...])
    m = jnp.max(logits, axis=-1, keepdims=True)
    ex = jnp.exp(logits - m)
    scores = ex / jnp.sum(ex, axis=-1, keepdims=True)

    eidx = lax.broadcasted_iota(jnp.int32, (_T, _E), 1)
    m0 = jnp.max(scores, axis=-1, keepdims=True)
    i0 = jnp.min(jnp.where(scores == m0, eidx, _E), axis=-1, keepdims=True)
    masked = jnp.where(eidx == i0, -jnp.inf, scores)
    m1 = jnp.max(masked, axis=-1, keepdims=True)
    i1 = jnp.min(jnp.where(masked == m1, eidx, _E), axis=-1, keepdims=True)

    denom = m0 + m1 + 1e-6
    g0_ref[...] = jnp.broadcast_to(m0 / denom, (_T, 16))
    g1_ref[...] = jnp.broadcast_to(m1 / denom, (_T, 16))

    oh0 = (eidx == i0).astype(jnp.float32)
    oh1 = (eidx == i1).astype(jnp.float32)
    ind = oh0 + oh1

    importance = jnp.sum(scores, axis=0, keepdims=True)
    load = jnp.sum(ind, axis=0, keepdims=True)
    aux_ref[...] = (_E / (_T * _T)) * jnp.sum(importance * load, axis=1,
                                              keepdims=True)

    c = ind
    k = 1
    while k < _T:
        c = c + jnp.concatenate([jnp.zeros((k, _E), jnp.float32), c[:-k, :]],
                                axis=0)
        k *= 2
    exc = c - ind
    cnt = c[_T - 1:_T, :]
    pad_cnt = jnp.ceil(cnt * (1.0 / _BT)) * _BT
    tri = (lax.broadcasted_iota(jnp.int32, (_E, _E), 0)
           < lax.broadcasted_iota(jnp.int32, (_E, _E), 1)).astype(jnp.float32)
    off = lax.dot_general(pad_cnt, tri, (((1,), (0,)), ((), ())))

    pos = off + exc
    p0_ref[...] = jnp.sum(oh0 * pos, axis=1, keepdims=True).astype(jnp.int32)
    p1_ref[...] = jnp.sum(oh1 * pos, axis=1, keepdims=True).astype(jnp.int32)

    bstart = (lax.broadcasted_iota(jnp.int32, (_NB, 1), 0) * _BT
              ).astype(jnp.float32)
    ge = (bstart >= off).astype(jnp.int32)
    eid_ref[...] = jnp.sum(ge, axis=1, keepdims=True) - 1

    xb16 = x.astype(jnp.bfloat16)
    lo = lax.bitcast_convert_type(xb16[:, :_D // 2], jnp.uint16)
    hi = lax.bitcast_convert_type(xb16[:, _D // 2:], jnp.uint16)
    xp_ref[...] = ((hi.astype(jnp.int32) << 16) | lo.astype(jnp.int32))


def _gate(x, gW1, gb1, gW2, gb2):
    out_shape = (
        jax.ShapeDtypeStruct((_T, 16), jnp.float32),
        jax.ShapeDtypeStruct((_T, 16), jnp.float32),
        jax.ShapeDtypeStruct((_T, 1), jnp.int32),
        jax.ShapeDtypeStruct((_T, 1), jnp.int32),
        jax.ShapeDtypeStruct((_NB, 1), jnp.int32),
        jax.ShapeDtypeStruct((1, 1), jnp.float32),
        jax.ShapeDtypeStruct((_T, _D // 2), jnp.int32),
    )
    return pl.pallas_call(
        _gate_kernel,
        out_shape=out_shape,
    )(x, gW1, gb1.reshape(1, -1), gW2, gb2.reshape(1, -1))


def _dispatch_body(x_hbm, idx_hbm, xs_hbm,
                   idx_v, buf_a, buf_b, sem_a, sem_b):
    wid = lax.axis_index("s") * 2 + lax.axis_index("c")
    base = wid * _TPW
    pltpu.sync_copy(idx_hbm.at[wid], idx_v)
    bufs = (buf_a, buf_b)
    sems = (sem_a, sem_b)
    pend = [None, None]
    for c in range(_DNC):
        p = c % 2
        if pend[p] is not None:
            pend[p][0].wait()
            pend[p][1].wait()
        pltpu.sync_copy(x_hbm.at[pl.ds(base + c * _DCH, _DCH)], bufs[p])
        cp0 = pltpu.async_copy(bufs[p], xs_hbm.at[idx_v.at[c]], sems[p])
        cp1 = pltpu.async_copy(bufs[p], xs_hbm.at[idx_v.at[_DNC + c]],
                               sems[p])
        pend[p] = (cp0, cp1)
    for p in range(2):
        if pend[p] is not None:
            pend[p][0].wait()
            pend[p][1].wait()


def _dispatch(x, idx4):
    mesh = plsc.VectorSubcoreMesh(core_axis_name="c", subcore_axis_name="s")
    f = pl.kernel(
        _dispatch_body,
        out_type=jax.ShapeDtypeStruct((_S, _D // 2), jnp.int32),
        mesh=mesh,
        scratch_types=[
            pltpu.VMEM((2 * _DNC, _DCH), jnp.int32),
            pltpu.VMEM((_DCH, _D // 2), jnp.int32),
            pltpu.VMEM((_DCH, _D // 2), jnp.int32),
            pltpu.SemaphoreType.DMA,
            pltpu.SemaphoreType.DMA,
        ],
    )
    return f(x, idx4)


def _group_kernel(eid_ref, xs_ref, Wa_ref, ba_ref, lg_ref, lb_ref,
                  Wb_ref, bb_ref, os_ref, wa_s, wb_s, last_s):
    i = pl.program_id(0)

    @pl.when(i == 0)
    def _():
        last_s[0] = -1

    e = eid_ref[i]

    @pl.when(last_s[0] != e)
    def _():
        wa_s[...] = Wa_ref[0].astype(jnp.bfloat16)
        wb_s[...] = Wb_ref[0].astype(jnp.bfloat16)
        last_s[0] = e

    w = xs_ref[...]
    xlo = lax.bitcast_convert_type((w & 0xFFFF).astype(jnp.uint16),
                                   jnp.bfloat16)
    xhi = lax.bitcast_convert_type(
        lax.shift_right_logical(w, 16).astype(jnp.uint16), jnp.bfloat16)
    xb = jnp.concatenate([xlo, xhi], axis=1)
    h = lax.dot_general(xb, wa_s[...], (((1,), (0,)), ((), ())),
                        preferred_element_type=jnp.float32) + ba_ref[0]
    mu = jnp.mean(h, axis=-1, keepdims=True)
    var = jnp.mean((h - mu) ** 2, axis=-1, keepdims=True)
    hn = (h - mu) * lax.rsqrt(var + 1e-5) * lg_ref[0] + lb_ref[0]
    ho = jnp.maximum(hn, 0.0).astype(jnp.bfloat16)
    eo = lax.dot_general(ho, wb_s[...], (((1,), (0,)), ((), ())),
                         preferred_element_type=jnp.float32) + bb_ref[0]
    os_ref[...] = eo


def _grouped(eid, xs, Wa_bf, ba3, lg3, lb3, Wb_bf, bb3):
    grid_spec = pltpu.PrefetchScalarGridSpec(
        num_scalar_prefetch=1,
        grid=(_NB,),
        in_specs=[
            pl.BlockSpec((_BT, _D // 2), lambda i, eid: (i, 0)),
            pl.BlockSpec((1, _D, _D), lambda i, eid: (eid[i], 0, 0)),
            pl.BlockSpec((1, 1, _D), lambda i, eid: (eid[i], 0, 0)),
            pl.BlockSpec((1, 1, _D), lambda i, eid: (eid[i], 0, 0)),
            pl.BlockSpec((1, 1, _D), lambda i, eid: (eid[i], 0, 0)),
            pl.BlockSpec((1, _D, _D), lambda i, eid: (eid[i], 0, 0)),
            pl.BlockSpec((1, 1, _D), lambda i, eid: (eid[i], 0, 0)),
        ],
        out_specs=pl.BlockSpec((_BT, _D), lambda i, eid: (i, 0)),
        scratch_shapes=[
            pltpu.VMEM((_D, _D), jnp.bfloat16),
            pltpu.VMEM((_D, _D), jnp.bfloat16),
            pltpu.SMEM((1,), jnp.int32),
        ],
    )
    return pl.pallas_call(
        _group_kernel,
        grid_spec=grid_spec,
        out_shape=jax.ShapeDtypeStruct((_S, _D), jnp.float32),
    )(eid, xs, Wa_bf, ba3, lg3, lb3, Wb_bf, bb3)


def _combine_body(os_hbm, idx_hbm, g0_hbm, g1_hbm, out_hbm,
                  idx_v, g0_v, g1_v, b0a, b1a, b0b, b1b,
                  s0a, s1a, s0b, s1b):
    wid = lax.axis_index("s") * 2 + lax.axis_index("c")
    base = wid * _TPW
    pltpu.sync_copy(idx_hbm.at[wid], idx_v)
    pltpu.sync_copy(g0_hbm.at[pl.ds(base, _TPW)], g0_v)
    pltpu.sync_copy(g1_hbm.at[pl.ds(base, _TPW)], g1_v)
    b0 = (b0a, b0b)
    b1 = (b1a, b1b)
    s0 = (s0a, s0b)
    s1 = (s1a, s1b)

    def gathers(c, p):
        cp0 = pltpu.async_copy(os_hbm.at[idx_v.at[c]], b0[p], s0[p])
        cp1 = pltpu.async_copy(os_hbm.at[idx_v.at[_CNC + c]], b1[p], s1[p])
        return cp0, cp1

    pend = gathers(0, 0)
    for c in range(_CNC):
        p = c % 2
        nxt = gathers(c + 1, 1 - p) if c + 1 < _CNC else None
        pend[0].wait()
        pend[1].wait()

        bb0 = b0[p]
        bb1 = b1[p]

        tok0 = c * _CCH

        def body(r, _):
            a = g0_v[tok0 + r]
            b = g1_v[tok0 + r]
            for j in range(_D // 16):
                sl = pl.ds(j * 16, 16)
                bb0[r, sl] = a * bb0[r, sl] + b * bb1[r, sl]
            return 0

        lax.fori_loop(0, _CCH, body, 0)
        pltpu.sync_copy(b0[p], out_hbm.at[pl.ds(base + c * _CCH, _CCH)])
        pend = nxt


def _combine(os, idx4c, g0r, g1r):
    mesh = plsc.VectorSubcoreMesh(core_axis_name="c", subcore_axis_name="s")
    f = pl.kernel(
        _combine_body,
        out_type=jax.ShapeDtypeStruct((_T, _D), jnp.float32),
        mesh=mesh,
        scratch_types=[
            pltpu.VMEM((2 * _CNC, _CCH), jnp.int32),
            pltpu.VMEM((_TPW, 16), jnp.float32),
            pltpu.VMEM((_TPW, 16), jnp.float32),
            pltpu.VMEM((_CCH, _D), jnp.float32),
            pltpu.VMEM((_CCH, _D), jnp.float32),
            pltpu.VMEM((_CCH, _D), jnp.float32),
            pltpu.VMEM((_CCH, _D), jnp.float32),
            pltpu.SemaphoreType.DMA,
            pltpu.SemaphoreType.DMA,
            pltpu.SemaphoreType.DMA,
            pltpu.SemaphoreType.DMA,
        ],
    )
    return f(os, idx4c, g0r, g1r)


def kernel(x, gW1, gb1, gW2, gb2, Wa, ba, lg, lb, Wb, bb):
    g0, g1, pos0, pos1, eid, aux, xp = _gate(x, gW1, gb1, gW2, gb2)

    idx4 = jnp.concatenate(
        [pos0.reshape(_NW, _DNC, _DCH), pos1.reshape(_NW, _DNC, _DCH)],
        axis=1)
    idx4c = jnp.concatenate(
        [pos0.reshape(_NW, _CNC, _CCH), pos1.reshape(_NW, _CNC, _CCH)],
        axis=1)

    xs = _dispatch(xp, idx4)
    os = _grouped(eid.reshape(_NB), xs,
                  Wa, ba.reshape(_E, 1, _D),
                  lg.reshape(_E, 1, _D), lb.reshape(_E, 1, _D),
                  Wb, bb.reshape(_E, 1, _D))
    out = _combine(os, idx4c, g0, g1)
    return out, aux[0, 0]
